# baseline (speedup 1.0000x reference)
import functools

import jax
import jax.numpy as jnp
from jax import lax
from jax.experimental import pallas as pl
from jax.experimental.pallas import tpu as pltpu
from jax.experimental.pallas import tpu_sc as plsc

_N = 10000
_E = 320000
_D = 128
_G = 64

_NC, _NS, _L = 2, 16, 16
_NW = _NC * _NS
_NR = 80
_NP = _NR * 128
_CHR = _NR // _NS
_CH = _CHR * 128

_NBLK_HI = 79
_NBLK_LO = 78
_EV = _NBLK_HI * 128
_DMA_MAX_BLK = _E // 128 - _NBLK_HI


def _sc_mesh():
    return plsc.VectorSubcoreMesh(
        core_axis_name="c", subcore_axis_name="s",
        num_cores=_NC, num_subcores=_NS)


def _zero_rows(ref, nrows):
    zeros = jnp.zeros((_L,), jnp.float32)

    def body(r, carry):
        for j in range(128 // _L):
            ref[r, pl.ds(j * _L, _L)] = zeros
        return carry

    lax.fori_loop(0, nrows, body, 0)


def _fill_iota(ref, n):
    lane = lax.iota(jnp.int32, _L)
    for j in range(n // _L):
        ref[pl.ds(j * _L, _L)] = lane + j * _L


def _edge_chunk(wid):
    base_blk = _NBLK_LO * wid + jnp.minimum(wid, 4)
    dma_blk = jnp.minimum(base_blk, _DMA_MAX_BLK)
    shift = (base_blk - dma_blk) * 128
    nblk = jnp.where(wid < 4, _NBLK_HI, _NBLK_LO)
    return dma_blk * 128, shift, nblk * (128 // _L)


def _split_idx(i16):
    return lax.shift_right_logical(i16, 7), jnp.bitwise_and(i16, 127)


@functools.cache
def _deg_kernel():
    @functools.partial(
        pl.kernel,
        out_type=jax.ShapeDtypeStruct((_NC, _NR, 128), jnp.float32),
        mesh=_sc_mesh(),
        compiler_params=pltpu.CompilerParams(needs_layout_passes=False),
        scratch_types=[
            pltpu.VMEM((2, _EV), jnp.int32),
            pltpu.VMEM((_NR, 128), jnp.float32),
            pltpu.VMEM_SHARED((_NS, _NR, 128), jnp.float32),
            pltpu.VMEM((_NS * 8, 128), jnp.float32),
            pltpu.VMEM((8, 128), jnp.float32),
            pltpu.SemaphoreType.DMA,
        ],
    )
    def deg(edge_hbm, zeros_hbm, out_hbm, ev, acc_v, slots_sh, red_v, out_v,
            sem):
        core = lax.axis_index("c")
        sid = lax.axis_index("s")
        wid = sid * _NC + core
        dma_base, shift, n_iters = _edge_chunk(wid)
        pltpu.sync_copy(edge_hbm.at[:, pl.ds(dma_base, _EV)], ev)
        pltpu.sync_copy(zeros_hbm, acc_v)
        ones = jnp.ones((_L,), jnp.float32)

        @plsc.parallel_loop(0, n_iters, unroll=8)
        def _(i):
            d16 = ev[1, pl.ds(shift + i * _L, _L)]
            r16, c16 = _split_idx(d16)
            plsc.addupdate_scatter(acc_v, [r16, c16], ones)
        pltpu.sync_copy(acc_v, slots_sh.at[sid])
        plsc.subcore_barrier()

        @pl.when(sid < _NR // 8)
        def _():
            copies = [
                pltpu.async_copy(
                    slots_sh.at[r, pl.ds(sid * 8, 8), :],
                    red_v.at[pl.ds(r * 8, 8), :], sem)
                for r in range(_NS)
            ]
            for c in copies:
                c.wait()

            def rsum(q, carry):
                rr = lax.shift_right_logical(q, 3)
                cc = pl.ds(jnp.bitwise_and(q, 7) * _L, _L)
                v = red_v[rr, cc]
                for r in range(1, _NS):
                    v = v + red_v[r * 8 + rr, cc]
                out_v[rr, cc] = v
                return carry

            lax.fori_loop(0, 8 * 128 // _L, rsum, 0)
            pltpu.sync_copy(out_v, out_hbm.at[core, pl.ds(sid * 8, 8), :])

    return deg


@functools.cache
def _msg_kernel():
    @functools.partial(
        pl.kernel,
        out_type=[
            jax.ShapeDtypeStruct((_NW, _G), jnp.float32),
            jax.ShapeDtypeStruct((_NW, _G), jnp.float32),
        ],
        mesh=_sc_mesh(),
        compiler_params=pltpu.CompilerParams(needs_layout_passes=False),
        scratch_types=[
            pltpu.VMEM((2, _EV), jnp.int32),
            pltpu.VMEM((_NP,), jnp.float32),
            pltpu.VMEM((_NR, 128), jnp.float32),
            pltpu.VMEM_SHARED((_NS, _NR, 128), jnp.float32),
            pltpu.VMEM((_NS * _CHR, 128), jnp.float32),
            pltpu.VMEM((_CHR, 128), jnp.float32),
            pltpu.VMEM((_CH,), jnp.float32),
            pltpu.VMEM((_CH,), jnp.float32),
            pltpu.VMEM((_CH,), jnp.int32),
            pltpu.VMEM((_G,), jnp.float32),
            pltpu.VMEM((_G,), jnp.float32),
            pltpu.SemaphoreType.DMA,
        ],
    )
    def msg(edge_hbm, zeros_hbm, zd_hbm, dinv_hbm, u_hbm, batch_hbm,
            pool_hbm, cnt_hbm,
            ev, zd_v, acc_v, slots_sh, red_v, t_c, dv_c, u_c, b_c,
            pool_v, cnt_v, sem):
        core = lax.axis_index("c")
        sid = lax.axis_index("s")
        wid = sid * _NC + core
        dma_base, shift, n_iters = _edge_chunk(wid)
        pltpu.sync_copy(edge_hbm.at[:, pl.ds(dma_base, _EV)], ev)
        pltpu.sync_copy(zd_hbm, zd_v)
        pltpu.sync_copy(zeros_hbm, acc_v)

        @plsc.parallel_loop(0, n_iters, unroll=8)
        def _(i):
            s16 = ev[0, pl.ds(shift + i * _L, _L)]
            d16 = ev[1, pl.ds(shift + i * _L, _L)]
            vals = plsc.load_gather(zd_v, [s16])
            r16, c16 = _split_idx(d16)
            plsc.addupdate_scatter(acc_v, [r16, c16], vals)
        pltpu.sync_copy(acc_v, slots_sh.at[sid])
        plsc.subcore_barrier()

        copies = [
            pltpu.async_copy(
                slots_sh.at[r, pl.ds(sid * _CHR, _CHR), :],
                red_v.at[pl.ds(r * _CHR, _CHR), :], sem)
            for r in range(_NS)
        ]
        for c in copies:
            c.wait()

        def rsum(q, carry):
            rr = lax.shift_right_logical(q, 3)
            cc = pl.ds(jnp.bitwise_and(q, 7) * _L, _L)
            v = red_v[rr, cc]
            for r in range(1, _NS):
                v = v + red_v[r * _CHR + rr, cc]
            t_c[rr, cc] = v
            return carry

        lax.fori_loop(0, _CHR * 128 // _L, rsum, 0)

        off = sid * _CH
        off_b = jnp.minimum(off, _N - _CH)
        bshift = off - off_b
        pltpu.sync_copy(dinv_hbm.at[pl.ds(off, _CH)], dv_c)
        pltpu.sync_copy(u_hbm.at[pl.ds(off, _CH)], u_c)
        pltpu.sync_copy(batch_hbm.at[pl.ds(off_b, _CH)], b_c)
        lane = lax.iota(jnp.int32, _L)
        zeros = jnp.zeros((_L,), jnp.float32)
        for j in range(_G // _L):
            pool_v[pl.ds(j * _L, _L)] = zeros
            cnt_v[pl.ds(j * _L, _L)] = zeros
        u_scale = jnp.where(core == 0, 1.0, 0.0).astype(jnp.float32)
        cnt16 = jnp.full((_L,), 1.0, jnp.float32) * u_scale

        def pstep(j, carry):
            idx = pl.ds((jnp.bitwise_and(j, 7)) * _L, _L)
            row = lax.shift_right_logical(j, 3)
            valid = (off + j * _L + lane) < _N
            s16 = dv_c[pl.ds(j * _L, _L)] * t_c[row, idx] \
                + u_c[pl.ds(j * _L, _L)] * u_scale
            b16 = b_c[pl.ds(bshift + j * _L, _L)]
            plsc.addupdate_scatter(pool_v, [b16], s16, mask=valid)
            plsc.addupdate_scatter(cnt_v, [b16], cnt16, mask=valid)
            return carry

        lax.fori_loop(0, _CH // _L, pstep, 0)
        pltpu.sync_copy(pool_v, pool_hbm.at[wid])
        pltpu.sync_copy(cnt_v, cnt_hbm.at[wid])

    return msg


def _dot(a, b, dims):
    return lax.dot_general(a, b, (dims, ((), ())),
                           preferred_element_type=jnp.float32,
                           precision=lax.Precision.HIGHEST)


def _prep_body(x_ref, w1_ref, wl_ref, degp_ref, zd_ref, dinv_ref, u_ref):
    w = _dot(w1_ref[...], wl_ref[...], ((1,), (0,)))
    z = _dot(w, x_ref[...], ((0,), (1,)))
    z = jnp.concatenate(
        [z, jnp.zeros((1, _NP - _N), jnp.float32)], axis=1)
    z = z.reshape(_NR, 128)
    deg = 1.0 + degp_ref[0] + degp_ref[1]
    dinv = 1.0 / jnp.sqrt(deg)
    zd_ref[...] = z * dinv
    dinv_ref[...] = dinv
    u_ref[...] = z / deg


_prep = pl.pallas_call(
    _prep_body,
    out_shape=[jax.ShapeDtypeStruct((_NR, 128), jnp.float32)] * 3,
)


def _finish_body(pool_ref, cnt_ref, b1r_ref, wl_ref, blr_ref, out_ref):
    psum = jnp.sum(pool_ref[...], axis=0, keepdims=True)
    cnt = jnp.sum(cnt_ref[...], axis=0, keepdims=True)
    c1 = _dot(b1r_ref[...], wl_ref[...], ((1,), (0,)))
    outr = (psum + cnt * c1) / jnp.maximum(cnt, 1.0) + blr_ref[...]
    out_ref[...] = outr.reshape(_G, 1)


_finish = pl.pallas_call(
    _finish_body,
    out_shape=jax.ShapeDtypeStruct((_G, 1), jnp.float32),
)


def kernel(x, edge_index, batch, W1, b1, Wl, bl):
    zeros = jnp.zeros((_NR, 128), jnp.float32)
    degp = _deg_kernel()(edge_index, zeros)
    zd, dinv, u = _prep(x, W1, Wl, degp)
    pool_p, cnt_p = _msg_kernel()(
        edge_index, zeros, zd.reshape(_NP), dinv.reshape(_NP),
        u.reshape(_NP), batch)
    out = _finish(pool_p, cnt_p, b1.reshape(1, _D), Wl, bl.reshape(1, 1))
    return out


# speedup vs baseline: 1.2058x; 1.2058x over previous
import functools

import jax
import jax.numpy as jnp
from jax import lax
from jax.experimental import pallas as pl
from jax.experimental.pallas import tpu as pltpu
from jax.experimental.pallas import tpu_sc as plsc

_N = 10000
_E = 320000
_D = 128
_G = 64

_NC, _NS, _L = 2, 16, 16
_NW = _NC * _NS
_NR = 80
_NP = _NR * 128
_CHR = _NR // _NS
_CH = _CHR * 128

_NBLK_HI = 79
_NBLK_LO = 78
_EV = _NBLK_HI * 128
_DMA_MAX_BLK = _E // 128 - _NBLK_HI


def _sc_mesh():
    return plsc.VectorSubcoreMesh(
        core_axis_name="c", subcore_axis_name="s",
        num_cores=_NC, num_subcores=_NS)


def _zero_rows(ref, nrows):
    zeros = jnp.zeros((_L,), jnp.float32)

    def body(r, carry):
        for j in range(128 // _L):
            ref[r, pl.ds(j * _L, _L)] = zeros
        return carry

    lax.fori_loop(0, nrows, body, 0)


def _fill_iota(ref, n):
    lane = lax.iota(jnp.int32, _L)
    for j in range(n // _L):
        ref[pl.ds(j * _L, _L)] = lane + j * _L


def _edge_chunk(wid):
    base_blk = _NBLK_LO * wid + jnp.minimum(wid, 4)
    dma_blk = jnp.minimum(base_blk, _DMA_MAX_BLK)
    shift = (base_blk - dma_blk) * 128
    nblk = jnp.where(wid < 4, _NBLK_HI, _NBLK_LO)
    return dma_blk * 128, shift, nblk * (128 // _L)


def _split_idx(i16):
    return lax.shift_right_logical(i16, 7), jnp.bitwise_and(i16, 127)


@functools.cache
def _deg_kernel():
    @functools.partial(
        pl.kernel,
        out_type=jax.ShapeDtypeStruct((_NC, _NR, 128), jnp.float32),
        mesh=_sc_mesh(),
        compiler_params=pltpu.CompilerParams(needs_layout_passes=False),
        scratch_types=[
            pltpu.VMEM((2, _EV), jnp.int32),
            pltpu.VMEM((_NR, 128), jnp.float32),
            pltpu.VMEM((_NR,), jnp.int32),
            pltpu.VMEM_SHARED((_NR, 128), jnp.float32),
        ],
    )
    def deg(edge_hbm, zeros_hbm, out_hbm, ev, acc_v, idx_v, deg_sh):
        core = lax.axis_index("c")
        sid = lax.axis_index("s")
        wid = sid * _NC + core
        dma_base, shift, n_iters = _edge_chunk(wid)
        pltpu.sync_copy(edge_hbm.at[:, pl.ds(dma_base, _EV)], ev)
        pltpu.sync_copy(zeros_hbm, acc_v)
        _fill_iota(idx_v, _NR)

        @pl.when(sid == 0)
        def _():
            pltpu.sync_copy(zeros_hbm, deg_sh)

        plsc.subcore_barrier()
        ones = jnp.ones((_L,), jnp.float32)

        @plsc.parallel_loop(0, n_iters, unroll=8)
        def _(i):
            d16 = ev[1, pl.ds(shift + i * _L, _L)]
            r16, c16 = _split_idx(d16)
            plsc.addupdate_scatter(acc_v, [r16, c16], ones)
        pltpu.sync_copy(acc_v, deg_sh.at[idx_v], add=True)
        plsc.subcore_barrier()

        @pl.when(sid == 0)
        def _():
            pltpu.sync_copy(deg_sh, out_hbm.at[core])

    return deg


@functools.cache
def _msg_kernel():
    @functools.partial(
        pl.kernel,
        out_type=[
            jax.ShapeDtypeStruct((_NW, _G), jnp.float32),
            jax.ShapeDtypeStruct((_NW, _G), jnp.float32),
        ],
        mesh=_sc_mesh(),
        compiler_params=pltpu.CompilerParams(needs_layout_passes=False),
        scratch_types=[
            pltpu.VMEM((2, _EV), jnp.int32),
            pltpu.VMEM((_NP,), jnp.float32),
            pltpu.VMEM((_NR, 128), jnp.float32),
            pltpu.VMEM((_NR,), jnp.int32),
            pltpu.VMEM_SHARED((_NR, 128), jnp.float32),
            pltpu.VMEM((_CHR, 128), jnp.float32),
            pltpu.VMEM((_CH,), jnp.float32),
            pltpu.VMEM((_CH,), jnp.float32),
            pltpu.VMEM((_CH,), jnp.int32),
            pltpu.VMEM((_G,), jnp.float32),
            pltpu.VMEM((_G,), jnp.float32),
        ],
    )
    def msg(edge_hbm, zeros_hbm, zd_hbm, dinv_hbm, u_hbm, batch_hbm,
            pool_hbm, cnt_hbm,
            ev, zd_v, acc_v, idx_v, t_sh, t_c, dv_c, u_c, b_c,
            pool_v, cnt_v):
        core = lax.axis_index("c")
        sid = lax.axis_index("s")
        wid = sid * _NC + core
        dma_base, shift, n_iters = _edge_chunk(wid)
        pltpu.sync_copy(edge_hbm.at[:, pl.ds(dma_base, _EV)], ev)
        pltpu.sync_copy(zd_hbm, zd_v)
        pltpu.sync_copy(zeros_hbm, acc_v)
        _fill_iota(idx_v, _NR)

        @pl.when(sid == 0)
        def _():
            pltpu.sync_copy(zeros_hbm, t_sh)

        plsc.subcore_barrier()

        @plsc.parallel_loop(0, n_iters, unroll=8)
        def _(i):
            s16 = ev[0, pl.ds(shift + i * _L, _L)]
            d16 = ev[1, pl.ds(shift + i * _L, _L)]
            vals = plsc.load_gather(zd_v, [s16])
            r16, c16 = _split_idx(d16)
            plsc.addupdate_scatter(acc_v, [r16, c16], vals)
        pltpu.sync_copy(acc_v, t_sh.at[idx_v], add=True)
        plsc.subcore_barrier()

        off = sid * _CH
        off_b = jnp.minimum(off, _N - _CH)
        bshift = off - off_b
        pltpu.sync_copy(t_sh.at[pl.ds(sid * _CHR, _CHR)], t_c)
        pltpu.sync_copy(dinv_hbm.at[pl.ds(off, _CH)], dv_c)
        pltpu.sync_copy(u_hbm.at[pl.ds(off, _CH)], u_c)
        pltpu.sync_copy(batch_hbm.at[pl.ds(off_b, _CH)], b_c)
        lane = lax.iota(jnp.int32, _L)
        zeros = jnp.zeros((_L,), jnp.float32)
        for j in range(_G // _L):
            pool_v[pl.ds(j * _L, _L)] = zeros
            cnt_v[pl.ds(j * _L, _L)] = zeros
        u_scale = jnp.where(core == 0, 1.0, 0.0).astype(jnp.float32)
        cnt16 = jnp.full((_L,), 1.0, jnp.float32) * u_scale

        def pstep(j, carry):
            idx = pl.ds((jnp.bitwise_and(j, 7)) * _L, _L)
            row = lax.shift_right_logical(j, 3)
            valid = (off + j * _L + lane) < _N
            s16 = dv_c[pl.ds(j * _L, _L)] * t_c[row, idx] \
                + u_c[pl.ds(j * _L, _L)] * u_scale
            b16 = b_c[pl.ds(bshift + j * _L, _L)]
            plsc.addupdate_scatter(pool_v, [b16], s16, mask=valid)
            plsc.addupdate_scatter(cnt_v, [b16], cnt16, mask=valid)
            return carry

        lax.fori_loop(0, _CH // _L, pstep, 0)
        pltpu.sync_copy(pool_v, pool_hbm.at[wid])
        pltpu.sync_copy(cnt_v, cnt_hbm.at[wid])

    return msg


def _dot(a, b, dims):
    return lax.dot_general(a, b, (dims, ((), ())),
                           preferred_element_type=jnp.float32,
                           precision=lax.Precision.HIGHEST)


def _prep_z_body(x_ref, w1_ref, wl_ref, z_ref):
    w = _dot(w1_ref[...], wl_ref[...], ((1,), (0,)))
    z = _dot(w, x_ref[...], ((0,), (1,)))
    z = jnp.concatenate(
        [z, jnp.zeros((1, _NP - _N), jnp.float32)], axis=1)
    z_ref[...] = z.reshape(_NR, 128)


_prep_z = pl.pallas_call(
    _prep_z_body,
    out_shape=jax.ShapeDtypeStruct((_NR, 128), jnp.float32),
)


def _prep_c_body(z_ref, degp_ref, zd_ref, dinv_ref, u_ref):
    z = z_ref[...]
    deg = 1.0 + degp_ref[0] + degp_ref[1]
    dinv = 1.0 / jnp.sqrt(deg)
    zd_ref[...] = z * dinv
    dinv_ref[...] = dinv
    u_ref[...] = z / deg


_prep_c = pl.pallas_call(
    _prep_c_body,
    out_shape=[jax.ShapeDtypeStruct((_NR, 128), jnp.float32)] * 3,
)


def _finish_body(pool_ref, cnt_ref, b1r_ref, wl_ref, blr_ref, out_ref):
    psum = jnp.sum(pool_ref[...], axis=0, keepdims=True)
    cnt = jnp.sum(cnt_ref[...], axis=0, keepdims=True)
    c1 = _dot(b1r_ref[...], wl_ref[...], ((1,), (0,)))
    outr = (psum + cnt * c1) / jnp.maximum(cnt, 1.0) + blr_ref[...]
    out_ref[...] = outr.reshape(_G, 1)


_finish = pl.pallas_call(
    _finish_body,
    out_shape=jax.ShapeDtypeStruct((_G, 1), jnp.float32),
)


def kernel(x, edge_index, batch, W1, b1, Wl, bl):
    zeros = jnp.zeros((_NR, 128), jnp.float32)
    z3 = _prep_z(x, W1, Wl)
    degp = _deg_kernel()(edge_index, zeros)
    zd, dinv, u = _prep_c(z3, degp)
    pool_p, cnt_p = _msg_kernel()(
        edge_index, zeros, zd.reshape(_NP), dinv.reshape(_NP),
        u.reshape(_NP), batch)
    out = _finish(pool_p, cnt_p, b1.reshape(1, _D), Wl, bl.reshape(1, 1))
    return out


# speedup vs baseline: 1.2953x; 1.0742x over previous
import functools

import jax
import jax.numpy as jnp
from jax import lax
from jax.experimental import pallas as pl
from jax.experimental.pallas import tpu as pltpu
from jax.experimental.pallas import tpu_sc as plsc

_N = 10000
_E = 320000
_D = 128
_G = 64

_NC, _NS, _L = 2, 16, 16
_NW = _NC * _NS
_NR = 80
_NP = _NR * 128
_CHR = _NR // _NS
_CH = _CHR * 128

_NBLK_HI = 79
_NBLK_LO = 78
_EV = _NBLK_HI * 128
_DMA_MAX_BLK = _E // 128 - _NBLK_HI


def _sc_mesh():
    return plsc.VectorSubcoreMesh(
        core_axis_name="c", subcore_axis_name="s",
        num_cores=_NC, num_subcores=_NS)


def _zero_rows(ref, nrows):
    zeros = jnp.zeros((_L,), jnp.float32)

    def body(r, carry):
        for j in range(128 // _L):
            ref[r, pl.ds(j * _L, _L)] = zeros
        return carry

    lax.fori_loop(0, nrows, body, 0)


def _fill_iota(ref, n):
    lane = lax.iota(jnp.int32, _L)
    for j in range(n // _L):
        ref[pl.ds(j * _L, _L)] = lane + j * _L


def _edge_chunk(wid):
    base_blk = _NBLK_LO * wid + jnp.minimum(wid, 4)
    dma_blk = jnp.minimum(base_blk, _DMA_MAX_BLK)
    shift = (base_blk - dma_blk) * 128
    nblk = jnp.where(wid < 4, _NBLK_HI, _NBLK_LO)
    return dma_blk * 128, shift, nblk * (128 // _L)


def _split_idx(i16):
    return lax.shift_right_logical(i16, 7), jnp.bitwise_and(i16, 127)


@functools.cache
def _deg_kernel():
    @functools.partial(
        pl.kernel,
        out_type=jax.ShapeDtypeStruct((_NC, _NR, 128), jnp.float32),
        mesh=_sc_mesh(),
        compiler_params=pltpu.CompilerParams(needs_layout_passes=False),
        scratch_types=[
            pltpu.VMEM((2, _EV), jnp.int32),
            pltpu.VMEM((_NR, 128), jnp.float32),
            pltpu.VMEM((_NR,), jnp.int32),
            pltpu.VMEM_SHARED((_NR, 128), jnp.float32),
        ],
    )
    def deg(edge_hbm, out_hbm, ev, acc_v, idx_v, deg_sh):
        core = lax.axis_index("c")
        sid = lax.axis_index("s")
        wid = sid * _NC + core
        dma_base, shift, n_iters = _edge_chunk(wid)
        pltpu.sync_copy(edge_hbm.at[:, pl.ds(dma_base, _EV)], ev)
        _zero_rows(acc_v, _NR)
        _fill_iota(idx_v, _NR)

        @pl.when(sid == 0)
        def _():
            pltpu.sync_copy(acc_v, deg_sh)

        plsc.subcore_barrier()
        ones = jnp.ones((_L,), jnp.float32)

        @plsc.parallel_loop(0, n_iters, unroll=8)
        def _(i):
            d16 = ev[1, pl.ds(shift + i * _L, _L)]
            r16, c16 = _split_idx(d16)
            plsc.addupdate_scatter(acc_v, [r16, c16], ones)
        pltpu.sync_copy(acc_v, deg_sh.at[idx_v], add=True)
        plsc.subcore_barrier()

        @pl.when(sid == 0)
        def _():
            pltpu.sync_copy(deg_sh, out_hbm.at[core])

    return deg


@functools.cache
def _msg_kernel():
    @functools.partial(
        pl.kernel,
        out_type=[
            jax.ShapeDtypeStruct((_NW, 128), jnp.float32),
            jax.ShapeDtypeStruct((_NW, 128), jnp.float32),
        ],
        mesh=_sc_mesh(),
        compiler_params=pltpu.CompilerParams(needs_layout_passes=False),
        scratch_types=[
            pltpu.VMEM((2, _EV), jnp.int32),
            pltpu.VMEM((_NP,), jnp.float32),
            pltpu.VMEM((_NR, 128), jnp.float32),
            pltpu.VMEM((_NR,), jnp.int32),
            pltpu.VMEM_SHARED((_NR, 128), jnp.float32),
            pltpu.VMEM((_CHR, 128), jnp.float32),
            pltpu.VMEM((_CH,), jnp.float32),
            pltpu.VMEM((_CH,), jnp.float32),
            pltpu.VMEM((_CH,), jnp.int32),
            pltpu.VMEM((128,), jnp.float32),
            pltpu.VMEM((128,), jnp.float32),
        ],
    )
    def msg(edge_hbm, zd_hbm, dinv_hbm, u_hbm, batch_hbm,
            pool_hbm, cnt_hbm,
            ev, zd_v, acc_v, idx_v, t_sh, t_c, dv_c, u_c, b_c,
            pool_v, cnt_v):
        core = lax.axis_index("c")
        sid = lax.axis_index("s")
        wid = sid * _NC + core
        dma_base, shift, n_iters = _edge_chunk(wid)
        pltpu.sync_copy(edge_hbm.at[:, pl.ds(dma_base, _EV)], ev)
        pltpu.sync_copy(zd_hbm, zd_v)
        _zero_rows(acc_v, _NR)
        _fill_iota(idx_v, _NR)

        @pl.when(sid == 0)
        def _():
            pltpu.sync_copy(acc_v, t_sh)

        plsc.subcore_barrier()

        @plsc.parallel_loop(0, n_iters, unroll=8)
        def _(i):
            s16 = ev[0, pl.ds(shift + i * _L, _L)]
            d16 = ev[1, pl.ds(shift + i * _L, _L)]
            vals = plsc.load_gather(zd_v, [s16])
            r16, c16 = _split_idx(d16)
            plsc.addupdate_scatter(acc_v, [r16, c16], vals)
        pltpu.sync_copy(acc_v, t_sh.at[idx_v], add=True)
        plsc.subcore_barrier()

        off = sid * _CH
        off_b = jnp.minimum(off, _N - _CH)
        bshift = off - off_b
        pltpu.sync_copy(t_sh.at[pl.ds(sid * _CHR, _CHR)], t_c)
        pltpu.sync_copy(dinv_hbm.at[pl.ds(off, _CH)], dv_c)
        pltpu.sync_copy(u_hbm.at[pl.ds(off, _CH)], u_c)
        pltpu.sync_copy(batch_hbm.at[pl.ds(off_b, _CH)], b_c)
        lane = lax.iota(jnp.int32, _L)
        zeros = jnp.zeros((_L,), jnp.float32)
        for j in range(128 // _L):
            pool_v[pl.ds(j * _L, _L)] = zeros
            cnt_v[pl.ds(j * _L, _L)] = zeros
        u_scale = jnp.where(core == 0, 1.0, 0.0).astype(jnp.float32)
        cnt16 = jnp.full((_L,), 1.0, jnp.float32) * u_scale

        def pstep(j, carry):
            idx = pl.ds((jnp.bitwise_and(j, 7)) * _L, _L)
            row = lax.shift_right_logical(j, 3)
            valid = (off + j * _L + lane) < _N
            s16 = dv_c[pl.ds(j * _L, _L)] * t_c[row, idx] \
                + u_c[pl.ds(j * _L, _L)] * u_scale
            b16 = b_c[pl.ds(bshift + j * _L, _L)]
            plsc.addupdate_scatter(pool_v, [b16], s16, mask=valid)
            plsc.addupdate_scatter(cnt_v, [b16], cnt16, mask=valid)
            return carry

        lax.fori_loop(0, _CH // _L, pstep, 0)
        pltpu.sync_copy(pool_v, pool_hbm.at[wid])
        pltpu.sync_copy(cnt_v, cnt_hbm.at[wid])

    return msg


def _dot(a, b, dims):
    return lax.dot_general(a, b, (dims, ((), ())),
                           preferred_element_type=jnp.float32,
                           precision=lax.Precision.HIGHEST)


def _prep_z_body(x_ref, w1_ref, wl_ref, z_ref):
    w = _dot(w1_ref[...], wl_ref[...], ((1,), (0,)))
    z = _dot(w, x_ref[...], ((0,), (1,)))
    z = jnp.concatenate(
        [z, jnp.zeros((1, _NP - _N), jnp.float32)], axis=1)
    z_ref[...] = z.reshape(_NR, 128)


_prep_z = pl.pallas_call(
    _prep_z_body,
    out_shape=jax.ShapeDtypeStruct((_NR, 128), jnp.float32),
)


def _prep_c_body(z_ref, degp_ref, zd_ref, dinv_ref, u_ref):
    z = z_ref[...]
    deg = 1.0 + degp_ref[0] + degp_ref[1]
    dinv = 1.0 / jnp.sqrt(deg)
    zd_ref[...] = z * dinv
    dinv_ref[...] = dinv
    u_ref[...] = z / deg


_prep_c = pl.pallas_call(
    _prep_c_body,
    out_shape=[jax.ShapeDtypeStruct((_NR, 128), jnp.float32)] * 3,
)


def _finish_body(pool_ref, cnt_ref, b1r_ref, wl_ref, blr_ref, out_ref):
    psum = jnp.sum(pool_ref[...], axis=0, keepdims=True)[:, :_G]
    cnt = jnp.sum(cnt_ref[...], axis=0, keepdims=True)[:, :_G]
    c1 = _dot(b1r_ref[...], wl_ref[...], ((1,), (0,)))
    outr = (psum + cnt * c1) / jnp.maximum(cnt, 1.0) + blr_ref[...]
    out_ref[...] = outr.reshape(_G, 1)


_finish = pl.pallas_call(
    _finish_body,
    out_shape=jax.ShapeDtypeStruct((_G, 1), jnp.float32),
)


def kernel(x, edge_index, batch, W1, b1, Wl, bl):
    z3 = _prep_z(x, W1, Wl)
    degp = _deg_kernel()(edge_index)
    zd, dinv, u = _prep_c(z3, degp)
    pool_p, cnt_p = _msg_kernel()(
        edge_index, zd.reshape(_NP), dinv.reshape(_NP),
        u.reshape(_NP), batch)
    out = _finish(pool_p, cnt_p, b1.reshape(1, _D), Wl, bl.reshape(1, 1))
    return out
